# Initial kernel scaffold; baseline (speedup 1.0000x reference)
import jax
import jax.numpy as jnp
from jax.experimental import pallas as pl


def kernel(vectors):
    raise NotImplementedError("write your pallas kernel here")


# speedup vs baseline: 4.4397x; 4.4397x over previous
import jax
import jax.numpy as jnp
from jax.experimental import pallas as pl
from jax.experimental.pallas import tpu as pltpu

SIZE = 128
STEEPNESS = 10.0
B_BLK = 32
RCHUNK = 8


def _w_layer(x, x_sw, is_lower, active):
    diff = x_sw - x
    t = diff * STEEPNESS / jnp.sqrt(jnp.sqrt(jnp.abs(diff) + 1e-10))
    s = jax.nn.sigmoid(t)
    w = jnp.where(is_lower, s, 1.0 - s)
    return jnp.where(active, w, 1.0)


def _diffsort_kernel(x_ref, xout_ref, mat_ref, scr_ref):
    n = SIZE
    x = x_ref[...]

    lane = jax.lax.broadcasted_iota(jnp.int32, (1, n), 1)
    even = lane % 2 == 0
    lowA = even
    actA = jnp.full_like(even, True)
    lowB = (~even) & (lane <= n - 3)
    actB = (lane >= 1) & (lane <= n - 2)

    row_id = jax.lax.broadcasted_iota(jnp.int32, (n, n), 0)
    col_id = jax.lax.broadcasted_iota(jnp.int32, (n, n), 1)
    eye = (row_id == col_id).astype(jnp.float32)
    mat_ref[...] = jnp.broadcast_to(eye[None], (B_BLK, n, n))

    def fused_pair(x, src_ref, dst_ref):
        x_swA = jnp.where(lowA, jnp.roll(x, -1, axis=-1),
                          jnp.roll(x, 1, axis=-1))
        w1 = _w_layer(x, x_swA, lowA, actA)
        x1 = x_swA + w1 * (x - x_swA)
        x_swB = jnp.where(lowB, jnp.roll(x1, -1, axis=-1),
                          jnp.roll(x1, 1, axis=-1))
        w2 = _w_layer(x1, x_swB, lowB, actB)
        x2 = x_swB + w2 * (x1 - x_swB)

        w1p2 = jnp.where(even, jnp.roll(w1, 1, axis=-1),
                         jnp.roll(w1, -1, axis=-1))
        a = w2 * w1
        b = w2 * (1.0 - w1)
        c = (1.0 - w2) * w1p2
        d = (1.0 - w2) * (1.0 - w1p2)
        zero = jnp.zeros_like(d)
        cm1 = jnp.where(even, b, c)[:, None, :]
        cp1 = jnp.where(even, c, b)[:, None, :]
        cm2 = jnp.where(even, zero, d)[:, None, :]
        cp2 = jnp.where(even, d, zero)[:, None, :]
        ca = a[:, None, :]

        def row_chunk(r, _):
            Xc = src_ref[:, pl.ds(r * RCHUNK, RCHUNK), :]
            acc = ca * Xc
            acc = acc + cm1 * pltpu.roll(Xc, n - 1, 2)
            acc = acc + cp1 * pltpu.roll(Xc, 1, 2)
            acc = acc + cm2 * pltpu.roll(Xc, n - 2, 2)
            acc = acc + cp2 * pltpu.roll(Xc, 2, 2)
            dst_ref[:, pl.ds(r * RCHUNK, RCHUNK), :] = acc
            return 0

        jax.lax.fori_loop(0, n // RCHUNK, row_chunk, 0)
        return x2

    def body(_, x):
        x = fused_pair(x, mat_ref, scr_ref)
        x = fused_pair(x, scr_ref, mat_ref)
        return x

    x = jax.lax.fori_loop(0, n // 4, body, x)
    xout_ref[...] = x


@jax.jit
def kernel(vectors):
    b, n = vectors.shape
    grid = (b // B_BLK,)
    xout, mat = pl.pallas_call(
        _diffsort_kernel,
        grid=grid,
        in_specs=[pl.BlockSpec((B_BLK, n), lambda i: (i, 0))],
        out_specs=[
            pl.BlockSpec((B_BLK, n), lambda i: (i, 0)),
            pl.BlockSpec((B_BLK, n, n), lambda i: (i, 0, 0)),
        ],
        out_shape=[
            jax.ShapeDtypeStruct((b, n), jnp.float32),
            jax.ShapeDtypeStruct((b, n, n), jnp.float32),
        ],
        scratch_shapes=[pltpu.VMEM((B_BLK, n, n), jnp.float32)],
        compiler_params=pltpu.CompilerParams(
            dimension_semantics=("parallel",),
        ),
    )(vectors)
    return xout, mat


# speedup vs baseline: 6.6526x; 1.4985x over previous
import jax
import jax.numpy as jnp
from jax.experimental import pallas as pl
from jax.experimental.pallas import tpu as pltpu

SIZE = 128
STEEPNESS = 10.0
B_BLK = 32
RCHUNK = 8


def _diffsort_kernel(x_ref, xout_ref, mat_ref, scr_ref):
    n = SIZE

    lane = jax.lax.broadcasted_iota(jnp.int32, (1, n), 1)
    even = lane % 2 == 0
    lowA = even
    actA = jnp.full_like(even, True)
    lowB = (~even) & (lane <= n - 3)
    actB = (lane >= 1) & (lane <= n - 2)

    def w_layer(x, x_sw, is_lower, active):
        diff = x_sw - x
        t = diff * STEEPNESS / jnp.sqrt(jnp.sqrt(jnp.abs(diff) + 1e-10))
        s = jax.nn.sigmoid(t)
        w = jnp.where(is_lower, s, 1.0 - s)
        return jnp.where(active, w, 1.0)

    def coeffs(x):
        x_swA = jnp.where(lowA, jnp.roll(x, -1, axis=-1),
                          jnp.roll(x, 1, axis=-1))
        w1 = w_layer(x, x_swA, lowA, actA)
        x1 = x_swA + w1 * (x - x_swA)
        x_swB = jnp.where(lowB, jnp.roll(x1, -1, axis=-1),
                          jnp.roll(x1, 1, axis=-1))
        w2 = w_layer(x1, x_swB, lowB, actB)
        x2 = x_swB + w2 * (x1 - x_swB)

        w1p2 = jnp.where(even, jnp.roll(w1, 1, axis=-1),
                         jnp.roll(w1, -1, axis=-1))
        a = w2 * w1
        b = w2 * (1.0 - w1)
        c = (1.0 - w2) * w1p2
        d = (1.0 - w2) * (1.0 - w1p2)
        zero = jnp.zeros_like(d)
        ca = a[:, None, :]
        cm1 = jnp.where(even, b, c)[:, None, :]
        cp1 = jnp.where(even, c, b)[:, None, :]
        cm2 = jnp.where(even, zero, d)[:, None, :]
        cp2 = jnp.where(even, d, zero)[:, None, :]
        return x2, (ca, cm1, cp1, cm2, cp2)

    def apply_step(C, src_ref, dst_ref):
        ca, cm1, cp1, cm2, cp2 = C
        for r in range(n // RCHUNK):
            Xc = src_ref[:, pl.ds(r * RCHUNK, RCHUNK), :]
            acc = ca * Xc
            acc = acc + cm1 * pltpu.roll(Xc, n - 1, 2)
            acc = acc + cp1 * pltpu.roll(Xc, 1, 2)
            acc = acc + cm2 * pltpu.roll(Xc, n - 2, 2)
            acc = acc + cp2 * pltpu.roll(Xc, 2, 2)
            dst_ref[:, pl.ds(r * RCHUNK, RCHUNK), :] = acc

    row_id = jax.lax.broadcasted_iota(jnp.int32, (n, n), 0)
    col_id = jax.lax.broadcasted_iota(jnp.int32, (n, n), 1)
    eye = (row_id == col_id).astype(jnp.float32)
    mat_ref[...] = jnp.broadcast_to(eye[None], (B_BLK, n, n))

    x0 = x_ref[...]
    x1, CA0 = coeffs(x0)
    x2, CB0 = coeffs(x1)

    def body(_, carry):
        xc, CA, CB = carry
        xn1, CA2 = coeffs(xc)
        xn2, CB2 = coeffs(xn1)
        apply_step(CA, mat_ref, scr_ref)
        apply_step(CB, scr_ref, mat_ref)
        return xn2, CA2, CB2

    xfin, CAl, CBl = jax.lax.fori_loop(0, n // 4 - 1, body, (x2, CA0, CB0))
    apply_step(CAl, mat_ref, scr_ref)
    apply_step(CBl, scr_ref, mat_ref)
    xout_ref[...] = xfin


@jax.jit
def kernel(vectors):
    b, n = vectors.shape
    grid = (b // B_BLK,)
    xout, mat = pl.pallas_call(
        _diffsort_kernel,
        grid=grid,
        in_specs=[pl.BlockSpec((B_BLK, n), lambda i: (i, 0))],
        out_specs=[
            pl.BlockSpec((B_BLK, n), lambda i: (i, 0)),
            pl.BlockSpec((B_BLK, n, n), lambda i: (i, 0, 0)),
        ],
        out_shape=[
            jax.ShapeDtypeStruct((b, n), jnp.float32),
            jax.ShapeDtypeStruct((b, n, n), jnp.float32),
        ],
        scratch_shapes=[pltpu.VMEM((B_BLK, n, n), jnp.float32)],
        compiler_params=pltpu.CompilerParams(
            dimension_semantics=("parallel",),
        ),
    )(vectors)
    return xout, mat
